# baseline (speedup 1.0000x reference)
import functools

import jax
import jax.numpy as jnp
from jax import lax
from jax.experimental import pallas as pl
from jax.experimental.pallas import tpu as pltpu
from jax.experimental.pallas import tpu_sc as plsc

N_USERS = 4000
N_MOVIES = 6000
N = N_USERS + N_MOVIES
N_PAD = 10240
HIDDEN = 512
OUT = 256
E = 160000
NT = 16
NSC = 2
BLK = 128
NBLK = 80
E_PAD = NT * NBLK * BLK
ROWS_PER_TILE = N_PAD // NT
R_BLK = 512
GRID = N_PAD // R_BLK

_MESH = plsc.VectorSubcoreMesh(core_axis_name="c", subcore_axis_name="s")


@functools.partial(
    pl.kernel,
    out_type=jax.ShapeDtypeStruct((NSC, N_PAD, BLK), jnp.float32),
    mesh=_MESH,
    scratch_types=[
        pltpu.VMEM((NBLK, 1, BLK), jnp.int32),
        pltpu.VMEM((BLK, BLK), jnp.float32),
        pltpu.VMEM_SHARED((N_PAD, BLK), jnp.float32),
        pltpu.SemaphoreType.DMA,
    ],
)
def _deg_kernel(dst_hbm, zeros_hbm, ones_hbm, deg_out, dslab, ones, acc, sem):
    c = lax.axis_index("c")
    w = lax.axis_index("s")
    rows = pl.ds(w * ROWS_PER_TILE, ROWS_PER_TILE)

    pltpu.sync_copy(ones_hbm, ones)
    pltpu.sync_copy(dst_hbm.at[w], dslab)
    pltpu.sync_copy(zeros_hbm.at[rows], acc.at[rows])
    plsc.subcore_barrier()

    half = NBLK // NSC
    base = c * half
    W = 8

    def body(j, _):
        pltpu.async_copy(ones, acc.at[dslab.at[base + j, 0]], sem, add=True)

        @pl.when(j >= W)
        def _():
            pltpu.make_async_copy(
                ones, acc.at[dslab.at[base + j - W, 0]], sem).wait()

        return 0

    lax.fori_loop(0, half, body, 0)

    def drain(j, _):
        pltpu.make_async_copy(
            ones, acc.at[dslab.at[base + half - W + j, 0]], sem).wait()
        return 0

    lax.fori_loop(0, W, drain, 0)
    plsc.subcore_barrier()
    pltpu.sync_copy(acc.at[rows], deg_out.at[c].at[rows])


HB = NBLK // 2


def _make_agg_kernel(nchunks):
    per_sc = nchunks // NSC

    def body(*refs):
        src_hbm, dst_hbm, zeros_hbm = refs[0], refs[1], refs[2]
        xt = refs[3:3 + nchunks]
        agg = refs[3 + nchunks:3 + 2 * nchunks]
        rest = refs[3 + 2 * nchunks:]
        sslab, dslab = rest[0], rest[1]
        rbuf = rest[2:4]
        acc = rest[4]
        gsem = rest[5:7]
        ssem = rest[7:9]

        c = lax.axis_index("c")
        w = lax.axis_index("s")
        rows = pl.ds(w * ROWS_PER_TILE, ROWS_PER_TILE)

        for ch in range(nchunks):
            owner = ch // per_sc

            @pl.when(c == owner)
            def _(ch=ch):
                xr = xt[ch]
                pltpu.sync_copy(zeros_hbm.at[rows], acc.at[rows])
                plsc.subcore_barrier()

                def fire(j, b):
                    pltpu.async_copy(
                        xr.at[sslab.at[j, 0]], rbuf[b], gsem[b])

                def scat(j, b):
                    pltpu.make_async_copy(
                        xr.at[sslab.at[j, 0]], rbuf[b], gsem[b]).wait()
                    pltpu.async_copy(
                        rbuf[b], acc.at[dslab.at[j, 0]], ssem[b], add=True)

                def sdrain(j, b):
                    pltpu.make_async_copy(
                        rbuf[b], acc.at[dslab.at[j, 0]], ssem[b]).wait()

                for h in range(NBLK // HB):
                    pltpu.sync_copy(
                        src_hbm.at[w, pl.ds(h * HB, HB)], sslab)
                    pltpu.sync_copy(
                        dst_hbm.at[w, pl.ds(h * HB, HB)], dslab)

                    def super_step(g, _):
                        j0 = 4 * g
                        fire(j0, 0)
                        fire(j0 + 1, 1)
                        scat(j0, 0)
                        scat(j0 + 1, 1)
                        sdrain(j0, 0)
                        fire(j0 + 2, 0)
                        sdrain(j0 + 1, 1)
                        fire(j0 + 3, 1)
                        scat(j0 + 2, 0)
                        scat(j0 + 3, 1)
                        sdrain(j0 + 2, 0)
                        sdrain(j0 + 3, 1)
                        return 0

                    lax.fori_loop(0, HB // 4, super_step, 0)
                plsc.subcore_barrier()
                pltpu.sync_copy(acc.at[rows], agg[ch].at[rows])

    shape = jax.ShapeDtypeStruct((N_PAD, BLK), jnp.float32)
    return pl.kernel(
        body,
        out_type=[shape] * nchunks,
        mesh=_MESH,
        scratch_types=(
            [pltpu.VMEM((HB, 1, BLK), jnp.int32)] * 2
            + [pltpu.VMEM((BLK, BLK), jnp.float32)] * 2
            + [pltpu.VMEM_SHARED((N_PAD, BLK), jnp.float32)]
            + [pltpu.SemaphoreType.DMA] * 4
        ),
    )


_agg4 = _make_agg_kernel(4)
_agg2 = _make_agg_kernel(2)


def _pre_body(x_ref, degp_ref, w_ref, dis_ref, xt0, xt1, xt2, xt3):
    deg = degp_ref[0, :, 0] + degp_ref[1, :, 0] + 1.0
    dis = lax.rsqrt(deg)
    h = jnp.dot(x_ref[...], w_ref[...], preferred_element_type=jnp.float32)
    xt = h * dis[:, None]
    dis_ref[...] = dis[:, None]
    for c, ref in enumerate((xt0, xt1, xt2, xt3)):
        ref[...] = xt[:, c * BLK:(c + 1) * BLK]


def _tc_pre(x_pad, deg_part, W1):
    chunk = jax.ShapeDtypeStruct((N_PAD, BLK), jnp.float32)
    return pl.pallas_call(
        _pre_body,
        grid=(GRID,),
        in_specs=[
            pl.BlockSpec((R_BLK, HIDDEN), lambda r: (r, 0)),
            pl.BlockSpec((NSC, R_BLK, BLK), lambda r: (0, r, 0)),
            pl.BlockSpec((HIDDEN, HIDDEN), lambda r: (0, 0)),
        ],
        out_specs=[
            pl.BlockSpec((R_BLK, 1), lambda r: (r, 0)),
            pl.BlockSpec((R_BLK, BLK), lambda r: (r, 0)),
            pl.BlockSpec((R_BLK, BLK), lambda r: (r, 0)),
            pl.BlockSpec((R_BLK, BLK), lambda r: (r, 0)),
            pl.BlockSpec((R_BLK, BLK), lambda r: (r, 0)),
        ],
        out_shape=[jax.ShapeDtypeStruct((N_PAD, 1), jnp.float32)] + [chunk] * 4,
    )(x_pad, deg_part, W1)


def _mid_body(a0, a1, a2, a3, x0, x1, x2, x3, dis_ref, b1_ref, w_ref,
              o0, o1):
    s = jnp.concatenate(
        [a[...] + x[...] for a, x in zip((a0, a1, a2, a3), (x0, x1, x2, x3))],
        axis=1)
    dis = dis_ref[...]
    y = jnp.maximum(s * dis + b1_ref[...][None, :], 0.0)
    h2 = jnp.dot(y, w_ref[...], preferred_element_type=jnp.float32)
    xt2 = h2 * dis
    o0[...] = xt2[:, :BLK]
    o1[...] = xt2[:, BLK:]


def _tc_mid(agg1, xt1, dis, b1, W2):
    chunk_spec = pl.BlockSpec((R_BLK, BLK), lambda r: (r, 0))
    chunk = jax.ShapeDtypeStruct((N_PAD, BLK), jnp.float32)
    return pl.pallas_call(
        _mid_body,
        grid=(GRID,),
        in_specs=[chunk_spec] * 8 + [
            pl.BlockSpec((R_BLK, 1), lambda r: (r, 0)),
            pl.BlockSpec((HIDDEN,), lambda r: (0,)),
            pl.BlockSpec((HIDDEN, OUT), lambda r: (0, 0)),
        ],
        out_specs=[chunk_spec, chunk_spec],
        out_shape=[chunk, chunk],
    )(*agg1, *xt1, dis, b1, W2)


def _fin_body(a0, a1, x0, x1, dis_ref, b2_ref, o_ref):
    s = jnp.concatenate([a0[...] + x0[...], a1[...] + x1[...]], axis=1)
    o_ref[...] = s * dis_ref[...] + b2_ref[...][None, :]


def _tc_fin(agg2, xt2, dis, b2):
    chunk_spec = pl.BlockSpec((R_BLK, BLK), lambda r: (r, 0))
    return pl.pallas_call(
        _fin_body,
        grid=(GRID,),
        in_specs=[chunk_spec] * 4 + [
            pl.BlockSpec((R_BLK, 1), lambda r: (r, 0)),
            pl.BlockSpec((OUT,), lambda r: (0,)),
        ],
        out_specs=pl.BlockSpec((R_BLK, OUT), lambda r: (r, 0)),
        out_shape=jax.ShapeDtypeStruct((N_PAD, OUT), jnp.float32),
    )(*agg2, *xt2, dis, b2)


def kernel(edge_index, user_emb, movie_emb, W1, b1, W2, b2):
    src = edge_index[0]
    dst = edge_index[1]
    pad = jnp.full((E_PAD - E,), N, dtype=jnp.int32)
    src_p = (jnp.concatenate([src, pad]).reshape(NBLK, NT, BLK)
             .transpose(1, 0, 2).reshape(NT, NBLK, 1, BLK))
    dst_p = (jnp.concatenate([dst, pad]).reshape(NBLK, NT, BLK)
             .transpose(1, 0, 2).reshape(NT, NBLK, 1, BLK))

    x_pad = jnp.zeros((N_PAD, HIDDEN), jnp.float32)
    x_pad = lax.dynamic_update_slice(x_pad, user_emb, (0, 0))
    x_pad = lax.dynamic_update_slice(x_pad, movie_emb, (N_USERS, 0))

    zeros128 = jnp.zeros((N_PAD, BLK), jnp.float32)
    ones128 = jnp.ones((BLK, BLK), jnp.float32)
    deg_part = _deg_kernel(dst_p, zeros128, ones128)
    dis, *xt1 = _tc_pre(x_pad, deg_part, W1)
    agg1 = _agg4(src_p, dst_p, zeros128, *xt1)
    xt2 = _tc_mid(agg1, xt1, dis, b1, W2)
    agg2 = _agg2(src_p, dst_p, zeros128, *xt2)
    out = _tc_fin(agg2, xt2, dis, b2)
    return out[:N]


# speedup vs baseline: 1.0855x; 1.0855x over previous
import functools

import jax
import jax.numpy as jnp
from jax import lax
from jax.experimental import pallas as pl
from jax.experimental.pallas import tpu as pltpu
from jax.experimental.pallas import tpu_sc as plsc

N_USERS = 4000
N_MOVIES = 6000
N = N_USERS + N_MOVIES
N_PAD = 10240
HIDDEN = 512
OUT = 256
E = 160000
NT = 16
NSC = 2
BLK = 128
NBLK = 80
E_PAD = NT * NBLK * BLK
ROWS_PER_TILE = N_PAD // NT
R_BLK = 512
GRID = N_PAD // R_BLK

_MESH = plsc.VectorSubcoreMesh(core_axis_name="c", subcore_axis_name="s")


@functools.partial(
    pl.kernel,
    out_type=jax.ShapeDtypeStruct((NSC, N_PAD, BLK), jnp.float32),
    mesh=_MESH,
    scratch_types=[
        pltpu.VMEM((NBLK, 1, BLK), jnp.int32),
        pltpu.VMEM((BLK, BLK), jnp.float32),
        pltpu.VMEM_SHARED((N_PAD, BLK), jnp.float32),
        pltpu.SemaphoreType.DMA,
    ],
)
def _deg_kernel(dst_hbm, zeros_hbm, ones_hbm, deg_out, dslab, ones, acc, sem):
    c = lax.axis_index("c")
    w = lax.axis_index("s")
    rows = pl.ds(w * ROWS_PER_TILE, ROWS_PER_TILE)

    pltpu.sync_copy(ones_hbm, ones)
    pltpu.sync_copy(dst_hbm.at[w], dslab)
    pltpu.sync_copy(zeros_hbm.at[rows], acc.at[rows])
    plsc.subcore_barrier()

    half = NBLK // NSC
    base = c * half
    W = 8

    def body(j, _):
        pltpu.async_copy(ones, acc.at[dslab.at[base + j, 0]], sem, add=True)

        @pl.when(j >= W)
        def _():
            pltpu.make_async_copy(
                ones, acc.at[dslab.at[base + j - W, 0]], sem).wait()

        return 0

    lax.fori_loop(0, half, body, 0)

    def drain(j, _):
        pltpu.make_async_copy(
            ones, acc.at[dslab.at[base + half - W + j, 0]], sem).wait()
        return 0

    lax.fori_loop(0, W, drain, 0)
    plsc.subcore_barrier()
    pltpu.sync_copy(acc.at[rows], deg_out.at[c].at[rows])


HB = NBLK // 2


def _make_agg_kernel(nchunks):
    per_sc = nchunks // NSC

    def body(*refs):
        src_hbm, dst_hbm, zeros_hbm = refs[0], refs[1], refs[2]
        xt = refs[3:3 + nchunks]
        agg = refs[3 + nchunks:3 + 2 * nchunks]
        rest = refs[3 + 2 * nchunks:]
        sslab, dslab = rest[0], rest[1]
        rbuf = rest[2:4]
        acc = rest[4]
        gsem = rest[5:7]

        c = lax.axis_index("c")
        w = lax.axis_index("s")
        rows = pl.ds(w * ROWS_PER_TILE, ROWS_PER_TILE)

        for ch in range(nchunks):
            owner = ch // per_sc

            @pl.when(c == owner)
            def _(ch=ch):
                xr = xt[ch]
                pltpu.sync_copy(zeros_hbm.at[rows], acc.at[rows])
                plsc.subcore_barrier()

                def fire(j, b):
                    pltpu.async_copy(
                        xr.at[sslab.at[j, 0]], rbuf[b], gsem[b])

                def scat(j, b):
                    pltpu.make_async_copy(
                        xr.at[sslab.at[j, 0]], rbuf[b], gsem[b]).wait()
                    pltpu.sync_copy(
                        rbuf[b], acc.at[dslab.at[j, 0]], add=True)

                for h in range(NBLK // HB):
                    pltpu.sync_copy(
                        src_hbm.at[w, pl.ds(h * HB, HB)], sslab)
                    pltpu.sync_copy(
                        dst_hbm.at[w, pl.ds(h * HB, HB)], dslab)

                    def super_step(g, _):
                        j0 = 8 * g
                        fire(j0, 0)
                        fire(j0 + 1, 1)
                        for k in range(6):
                            scat(j0 + k, k % 2)
                            fire(j0 + k + 2, k % 2)
                        scat(j0 + 6, 0)
                        scat(j0 + 7, 1)
                        return 0

                    lax.fori_loop(0, HB // 8, super_step, 0)
                plsc.subcore_barrier()
                pltpu.sync_copy(acc.at[rows], agg[ch].at[rows])

    shape = jax.ShapeDtypeStruct((N_PAD, BLK), jnp.float32)
    return pl.kernel(
        body,
        out_type=[shape] * nchunks,
        mesh=_MESH,
        scratch_types=(
            [pltpu.VMEM((HB, 1, BLK), jnp.int32)] * 2
            + [pltpu.VMEM((BLK, BLK), jnp.float32)] * 2
            + [pltpu.VMEM_SHARED((N_PAD, BLK), jnp.float32)]
            + [pltpu.SemaphoreType.DMA] * 2
        ),
    )


_agg4 = _make_agg_kernel(4)
_agg2 = _make_agg_kernel(2)


def _pre_body(x_ref, degp_ref, w_ref, dis_ref, xt0, xt1, xt2, xt3):
    deg = degp_ref[0, :, 0] + degp_ref[1, :, 0] + 1.0
    dis = lax.rsqrt(deg)
    h = jnp.dot(x_ref[...], w_ref[...], preferred_element_type=jnp.float32)
    xt = h * dis[:, None]
    dis_ref[...] = dis[:, None]
    for c, ref in enumerate((xt0, xt1, xt2, xt3)):
        ref[...] = xt[:, c * BLK:(c + 1) * BLK]


def _tc_pre(x_pad, deg_part, W1):
    chunk = jax.ShapeDtypeStruct((N_PAD, BLK), jnp.float32)
    return pl.pallas_call(
        _pre_body,
        grid=(GRID,),
        in_specs=[
            pl.BlockSpec((R_BLK, HIDDEN), lambda r: (r, 0)),
            pl.BlockSpec((NSC, R_BLK, BLK), lambda r: (0, r, 0)),
            pl.BlockSpec((HIDDEN, HIDDEN), lambda r: (0, 0)),
        ],
        out_specs=[
            pl.BlockSpec((R_BLK, 1), lambda r: (r, 0)),
            pl.BlockSpec((R_BLK, BLK), lambda r: (r, 0)),
            pl.BlockSpec((R_BLK, BLK), lambda r: (r, 0)),
            pl.BlockSpec((R_BLK, BLK), lambda r: (r, 0)),
            pl.BlockSpec((R_BLK, BLK), lambda r: (r, 0)),
        ],
        out_shape=[jax.ShapeDtypeStruct((N_PAD, 1), jnp.float32)] + [chunk] * 4,
    )(x_pad, deg_part, W1)


def _mid_body(a0, a1, a2, a3, x0, x1, x2, x3, dis_ref, b1_ref, w_ref,
              o0, o1):
    s = jnp.concatenate(
        [a[...] + x[...] for a, x in zip((a0, a1, a2, a3), (x0, x1, x2, x3))],
        axis=1)
    dis = dis_ref[...]
    y = jnp.maximum(s * dis + b1_ref[...][None, :], 0.0)
    h2 = jnp.dot(y, w_ref[...], preferred_element_type=jnp.float32)
    xt2 = h2 * dis
    o0[...] = xt2[:, :BLK]
    o1[...] = xt2[:, BLK:]


def _tc_mid(agg1, xt1, dis, b1, W2):
    chunk_spec = pl.BlockSpec((R_BLK, BLK), lambda r: (r, 0))
    chunk = jax.ShapeDtypeStruct((N_PAD, BLK), jnp.float32)
    return pl.pallas_call(
        _mid_body,
        grid=(GRID,),
        in_specs=[chunk_spec] * 8 + [
            pl.BlockSpec((R_BLK, 1), lambda r: (r, 0)),
            pl.BlockSpec((HIDDEN,), lambda r: (0,)),
            pl.BlockSpec((HIDDEN, OUT), lambda r: (0, 0)),
        ],
        out_specs=[chunk_spec, chunk_spec],
        out_shape=[chunk, chunk],
    )(*agg1, *xt1, dis, b1, W2)


def _fin_body(a0, a1, x0, x1, dis_ref, b2_ref, o_ref):
    s = jnp.concatenate([a0[...] + x0[...], a1[...] + x1[...]], axis=1)
    o_ref[...] = s * dis_ref[...] + b2_ref[...][None, :]


def _tc_fin(agg2, xt2, dis, b2):
    chunk_spec = pl.BlockSpec((R_BLK, BLK), lambda r: (r, 0))
    return pl.pallas_call(
        _fin_body,
        grid=(GRID,),
        in_specs=[chunk_spec] * 4 + [
            pl.BlockSpec((R_BLK, 1), lambda r: (r, 0)),
            pl.BlockSpec((OUT,), lambda r: (0,)),
        ],
        out_specs=pl.BlockSpec((R_BLK, OUT), lambda r: (r, 0)),
        out_shape=jax.ShapeDtypeStruct((N_PAD, OUT), jnp.float32),
    )(*agg2, *xt2, dis, b2)


def kernel(edge_index, user_emb, movie_emb, W1, b1, W2, b2):
    src = edge_index[0]
    dst = edge_index[1]
    pad = jnp.full((E_PAD - E,), N, dtype=jnp.int32)
    src_p = (jnp.concatenate([src, pad]).reshape(NBLK, NT, BLK)
             .transpose(1, 0, 2).reshape(NT, NBLK, 1, BLK))
    dst_p = (jnp.concatenate([dst, pad]).reshape(NBLK, NT, BLK)
             .transpose(1, 0, 2).reshape(NT, NBLK, 1, BLK))

    x_pad = jnp.zeros((N_PAD, HIDDEN), jnp.float32)
    x_pad = lax.dynamic_update_slice(x_pad, user_emb, (0, 0))
    x_pad = lax.dynamic_update_slice(x_pad, movie_emb, (N_USERS, 0))

    zeros128 = jnp.zeros((N_PAD, BLK), jnp.float32)
    ones128 = jnp.ones((BLK, BLK), jnp.float32)
    deg_part = _deg_kernel(dst_p, zeros128, ones128)
    dis, *xt1 = _tc_pre(x_pad, deg_part, W1)
    agg1 = _agg4(src_p, dst_p, zeros128, *xt1)
    xt2 = _tc_mid(agg1, xt1, dis, b1, W2)
    agg2 = _agg2(src_p, dst_p, zeros128, *xt2)
    out = _tc_fin(agg2, xt2, dis, b2)
    return out[:N]


# speedup vs baseline: 1.1142x; 1.0265x over previous
import functools

import jax
import jax.numpy as jnp
from jax import lax
from jax.experimental import pallas as pl
from jax.experimental.pallas import tpu as pltpu
from jax.experimental.pallas import tpu_sc as plsc

N_USERS = 4000
N_MOVIES = 6000
N = N_USERS + N_MOVIES
N_PAD = 10240
HIDDEN = 512
OUT = 256
E = 160000
NT = 16
NSC = 2
BLK = 128
NBLK = 80
E_PAD = NT * NBLK * BLK
ROWS_PER_TILE = N_PAD // NT
R_BLK = 512
GRID = N_PAD // R_BLK

_MESH = plsc.VectorSubcoreMesh(core_axis_name="c", subcore_axis_name="s")


@functools.partial(
    pl.kernel,
    out_type=jax.ShapeDtypeStruct((NSC, N_PAD, BLK), jnp.float32),
    mesh=_MESH,
    scratch_types=[
        pltpu.VMEM((NBLK, 1, BLK), jnp.int32),
        pltpu.VMEM((BLK, BLK), jnp.float32),
        pltpu.VMEM_SHARED((N_PAD, BLK), jnp.float32),
        pltpu.SemaphoreType.DMA,
    ],
)
def _deg_kernel(dst_hbm, zeros_hbm, ones_hbm, deg_out, dslab, ones, acc, sem):
    c = lax.axis_index("c")
    w = lax.axis_index("s")
    rows = pl.ds(w * ROWS_PER_TILE, ROWS_PER_TILE)

    pltpu.sync_copy(ones_hbm, ones)
    pltpu.sync_copy(dst_hbm.at[w], dslab)
    pltpu.sync_copy(zeros_hbm.at[rows], acc.at[rows])
    plsc.subcore_barrier()

    half = NBLK // NSC
    base = c * half
    W = 8

    def body(j, _):
        pltpu.async_copy(ones, acc.at[dslab.at[base + j, 0]], sem, add=True)

        @pl.when(j >= W)
        def _():
            pltpu.make_async_copy(
                ones, acc.at[dslab.at[base + j - W, 0]], sem).wait()

        return 0

    lax.fori_loop(0, half, body, 0)

    def drain(j, _):
        pltpu.make_async_copy(
            ones, acc.at[dslab.at[base + half - W + j, 0]], sem).wait()
        return 0

    lax.fori_loop(0, W, drain, 0)
    plsc.subcore_barrier()
    pltpu.sync_copy(acc.at[rows], deg_out.at[c].at[rows])


HB = NBLK // 2


def _make_agg_kernel(nchunks):
    per_sc = nchunks // NSC

    def body(*refs):
        src_hbm, dst_hbm, zeros_hbm = refs[0], refs[1], refs[2]
        xt = refs[3:3 + nchunks]
        agg = refs[3 + nchunks:3 + 2 * nchunks]
        rest = refs[3 + 2 * nchunks:]
        sslab, dslab = rest[0], rest[1]
        rbuf = rest[2:4]
        acc = rest[4]
        gsem = rest[5:7]

        c = lax.axis_index("c")
        w = lax.axis_index("s")
        rows = pl.ds(w * ROWS_PER_TILE, ROWS_PER_TILE)

        for ch in range(nchunks):
            owner = ch // per_sc

            @pl.when(c == owner)
            def _(ch=ch):
                xr = xt[ch]
                pltpu.sync_copy(zeros_hbm.at[rows], acc.at[rows])
                plsc.subcore_barrier()

                def fire(j, b):
                    pltpu.async_copy(
                        xr.at[sslab.at[j, 0]], rbuf[b], gsem[b])

                def scat(j, b):
                    pltpu.make_async_copy(
                        xr.at[sslab.at[j, 0]], rbuf[b], gsem[b]).wait()
                    pltpu.sync_copy(
                        rbuf[b], acc.at[dslab.at[j, 0]], add=True)

                for h in range(NBLK // HB):
                    pltpu.sync_copy(
                        src_hbm.at[w, pl.ds(h * HB, HB)], sslab)
                    pltpu.sync_copy(
                        dst_hbm.at[w, pl.ds(h * HB, HB)], dslab)

                    def super_step(g, _):
                        j0 = 20 * g
                        fire(j0, 0)
                        fire(j0 + 1, 1)
                        for k in range(18):
                            scat(j0 + k, k % 2)
                            fire(j0 + k + 2, k % 2)
                        scat(j0 + 18, 0)
                        scat(j0 + 19, 1)
                        return 0

                    lax.fori_loop(0, HB // 20, super_step, 0)
                plsc.subcore_barrier()
                pltpu.sync_copy(acc.at[rows], agg[ch].at[rows])

    shape = jax.ShapeDtypeStruct((N_PAD, BLK), jnp.float32)
    return pl.kernel(
        body,
        out_type=[shape] * nchunks,
        mesh=_MESH,
        scratch_types=(
            [pltpu.VMEM((HB, 1, BLK), jnp.int32)] * 2
            + [pltpu.VMEM((BLK, BLK), jnp.float32)] * 2
            + [pltpu.VMEM_SHARED((N_PAD, BLK), jnp.float32)]
            + [pltpu.SemaphoreType.DMA] * 2
        ),
    )


_agg4 = _make_agg_kernel(4)
_agg2 = _make_agg_kernel(2)


def _pre_body(x_ref, degp_ref, w_ref, dis_ref, xt0, xt1, xt2, xt3):
    deg = degp_ref[0, :, 0] + degp_ref[1, :, 0] + 1.0
    dis = lax.rsqrt(deg)
    h = jnp.dot(x_ref[...], w_ref[...], preferred_element_type=jnp.float32)
    xt = h * dis[:, None]
    dis_ref[...] = dis[:, None]
    for c, ref in enumerate((xt0, xt1, xt2, xt3)):
        ref[...] = xt[:, c * BLK:(c + 1) * BLK]


def _tc_pre(x_pad, deg_part, W1):
    chunk = jax.ShapeDtypeStruct((N_PAD, BLK), jnp.float32)
    return pl.pallas_call(
        _pre_body,
        grid=(GRID,),
        in_specs=[
            pl.BlockSpec((R_BLK, HIDDEN), lambda r: (r, 0)),
            pl.BlockSpec((NSC, R_BLK, BLK), lambda r: (0, r, 0)),
            pl.BlockSpec((HIDDEN, HIDDEN), lambda r: (0, 0)),
        ],
        out_specs=[
            pl.BlockSpec((R_BLK, 1), lambda r: (r, 0)),
            pl.BlockSpec((R_BLK, BLK), lambda r: (r, 0)),
            pl.BlockSpec((R_BLK, BLK), lambda r: (r, 0)),
            pl.BlockSpec((R_BLK, BLK), lambda r: (r, 0)),
            pl.BlockSpec((R_BLK, BLK), lambda r: (r, 0)),
        ],
        out_shape=[jax.ShapeDtypeStruct((N_PAD, 1), jnp.float32)] + [chunk] * 4,
    )(x_pad, deg_part, W1)


def _mid_body(a0, a1, a2, a3, x0, x1, x2, x3, dis_ref, b1_ref, w_ref,
              o0, o1):
    s = jnp.concatenate(
        [a[...] + x[...] for a, x in zip((a0, a1, a2, a3), (x0, x1, x2, x3))],
        axis=1)
    dis = dis_ref[...]
    y = jnp.maximum(s * dis + b1_ref[...][None, :], 0.0)
    h2 = jnp.dot(y, w_ref[...], preferred_element_type=jnp.float32)
    xt2 = h2 * dis
    o0[...] = xt2[:, :BLK]
    o1[...] = xt2[:, BLK:]


def _tc_mid(agg1, xt1, dis, b1, W2):
    chunk_spec = pl.BlockSpec((R_BLK, BLK), lambda r: (r, 0))
    chunk = jax.ShapeDtypeStruct((N_PAD, BLK), jnp.float32)
    return pl.pallas_call(
        _mid_body,
        grid=(GRID,),
        in_specs=[chunk_spec] * 8 + [
            pl.BlockSpec((R_BLK, 1), lambda r: (r, 0)),
            pl.BlockSpec((HIDDEN,), lambda r: (0,)),
            pl.BlockSpec((HIDDEN, OUT), lambda r: (0, 0)),
        ],
        out_specs=[chunk_spec, chunk_spec],
        out_shape=[chunk, chunk],
    )(*agg1, *xt1, dis, b1, W2)


def _fin_body(a0, a1, x0, x1, dis_ref, b2_ref, o_ref):
    s = jnp.concatenate([a0[...] + x0[...], a1[...] + x1[...]], axis=1)
    o_ref[...] = s * dis_ref[...] + b2_ref[...][None, :]


def _tc_fin(agg2, xt2, dis, b2):
    chunk_spec = pl.BlockSpec((R_BLK, BLK), lambda r: (r, 0))
    return pl.pallas_call(
        _fin_body,
        grid=(GRID,),
        in_specs=[chunk_spec] * 4 + [
            pl.BlockSpec((R_BLK, 1), lambda r: (r, 0)),
            pl.BlockSpec((OUT,), lambda r: (0,)),
        ],
        out_specs=pl.BlockSpec((R_BLK, OUT), lambda r: (r, 0)),
        out_shape=jax.ShapeDtypeStruct((N_PAD, OUT), jnp.float32),
    )(*agg2, *xt2, dis, b2)


def kernel(edge_index, user_emb, movie_emb, W1, b1, W2, b2):
    src = edge_index[0]
    dst = edge_index[1]
    pad = jnp.full((E_PAD - E,), N, dtype=jnp.int32)
    src_p = (jnp.concatenate([src, pad]).reshape(NBLK, NT, BLK)
             .transpose(1, 0, 2).reshape(NT, NBLK, 1, BLK))
    dst_p = (jnp.concatenate([dst, pad]).reshape(NBLK, NT, BLK)
             .transpose(1, 0, 2).reshape(NT, NBLK, 1, BLK))

    x_pad = jnp.zeros((N_PAD, HIDDEN), jnp.float32)
    x_pad = lax.dynamic_update_slice(x_pad, user_emb, (0, 0))
    x_pad = lax.dynamic_update_slice(x_pad, movie_emb, (N_USERS, 0))

    zeros128 = jnp.zeros((N_PAD, BLK), jnp.float32)
    ones128 = jnp.ones((BLK, BLK), jnp.float32)
    deg_part = _deg_kernel(dst_p, zeros128, ones128)
    dis, *xt1 = _tc_pre(x_pad, deg_part, W1)
    agg1 = _agg4(src_p, dst_p, zeros128, *xt1)
    xt2 = _tc_mid(agg1, xt1, dis, b1, W2)
    agg2 = _agg2(src_p, dst_p, zeros128, *xt2)
    out = _tc_fin(agg2, xt2, dis, b2)
    return out[:N]
